# baseline (speedup 1.0000x reference)
import functools

import jax
import jax.numpy as jnp
from jax import lax
from jax.experimental import pallas as pl
from jax.experimental.pallas import tpu as pltpu
from jax.experimental.pallas import tpu_sc as plsc

N = 10000
C = 128
E = 320000

NC = 2
NS = 16
NW = NC * NS
K = 112
CHUNKS = 90
DEPTH = 3
EP = CHUNKS * K
E_PAD = NW * EP
NPE = E_PAD - E


def _leaky(e):
    return jnp.where(e > 0, e, 0.2 * e)


def _edge_body(h_h, as_h, ad_h, sd_h, z2d_h, z1d_h, acc_out, s_out,
               acc_sh, s_sh, rows0, rows1, rows2,
               sd0, sd1, sd2, sd3, sd4, sd5,
               asb0, asb1, asb2, adb0, adb1, adb2,
               pb0, pb1, pb2,
               semi0, semi1, semi2, semi3, semi4, semi5,
               semr0, semr1, semr2,
               semw0, semw1, semw2, sems0, sems1, sems2):
    cid = lax.axis_index("c")
    sid = lax.axis_index("s")
    wid = sid * NC + cid
    row0 = wid * CHUNKS

    rows_b = (rows0, rows1, rows2)
    sd_b = (sd0, sd1, sd2, sd3, sd4, sd5)
    asb_b = (asb0, asb1, asb2)
    adb_b = (adb0, adb1, adb2)
    pb_b = (pb0, pb1, pb2)
    semi_b = (semi0, semi1, semi2, semi3, semi4, semi5)
    semr_b = (semr0, semr1, semr2)
    semw_b = (semw0, semw1, semw2)
    sems_b = (sems0, sems1, sems2)

    @pl.when(sid < 15)
    def _zacc_main():
        pltpu.sync_copy(z2d_h, acc_sh.at[pl.ds(sid * 640, 640)])

    @pl.when(sid == 15)
    def _zacc_tail():
        pltpu.sync_copy(z2d_h.at[pl.ds(0, 400)], acc_sh.at[pl.ds(9600, 400)])

    pltpu.sync_copy(z1d_h.at[pl.ds(sid * 640, 640)],
                    s_sh.at[pl.ds(sid * 640, 640)])

    def _issue_idx(ci, b):
        pltpu.async_copy(sd_h.at[row0 + ci], sd_b[b], semi_b[b])

    def _wait_idx(ci, b):
        pltpu.make_async_copy(sd_h.at[row0 + ci], sd_b[b], semi_b[b]).wait()

    def _issue_gathers(ci, islot, b):
        pltpu.async_copy(as_h.at[sd_b[islot].at[0]], asb_b[b], semr_b[b])
        pltpu.async_copy(ad_h.at[sd_b[islot].at[1]], adb_b[b], semr_b[b])
        pltpu.async_copy(h_h.at[sd_b[islot].at[0]], rows_b[b], semr_b[b])

    def _wait_gathers(islot, b):
        pltpu.make_async_copy(as_h.at[sd_b[islot].at[0]], asb_b[b],
                              semr_b[b]).wait()
        pltpu.make_async_copy(ad_h.at[sd_b[islot].at[1]], adb_b[b],
                              semr_b[b]).wait()
        pltpu.make_async_copy(h_h.at[sd_b[islot].at[0]], rows_b[b],
                              semr_b[b]).wait()

    def _wait_scatters(islot, b):
        pltpu.make_async_copy(rows_b[b], acc_sh.at[sd_b[islot].at[1]],
                              semw_b[b]).wait()
        pltpu.make_async_copy(pb_b[b], s_sh.at[sd_b[islot].at[1]],
                              sems_b[b]).wait()

    plsc.subcore_barrier()

    for s0 in range(6):
        _issue_idx(s0, s0)
    _wait_idx(0, 0)
    _issue_gathers(0, 0, 0)

    def ring(g, _):
        for u in range(6):
            ci = g * 6 + u
            islot = u
            b = u % DEPTH
            b1 = (b + 1) % DEPTH

            islot1 = (u + 1) % 6

            @pl.when(jnp.logical_and(ci >= 2, ci + 1 < CHUNKS))
            def _drain_next():
                _wait_scatters((islot1 + 3) % 6, b1)

            @pl.when(ci + 1 < CHUNKS)
            def _feed():
                _wait_idx(ci + 1, islot1)
                _issue_gathers(ci + 1, islot1, b1)

            _wait_gathers(islot, b)
            rows, pb, sd = rows_b[b], pb_b[b], sd_b[islot]
            for j in range(K // 16):
                e = (asb_b[b][pl.ds(j * 16, 16)]
                     + adb_b[b][pl.ds(j * 16, 16)])
                p = jnp.exp(_leaky(e))
                pb[pl.ds(j * 16, 16)] = p
            pltpu.make_async_copy(pb, s_sh.at[sd.at[1]],
                                  sems_b[b]).start(add=True)

            def scale(g2, _2):
                pvec = pb[pl.ds(g2 * 16, 16)]
                for i in range(16):
                    pe = pvec[i]
                    ei = g2 * 16 + i
                    for k2 in range(8):
                        rows[ei, pl.ds(k2 * 16, 16)] = (
                            rows[ei, pl.ds(k2 * 16, 16)] * pe)
                return _2
            lax.fori_loop(0, K // 16, scale, 0)
            pltpu.make_async_copy(rows, acc_sh.at[sd.at[1]],
                                  semw_b[b]).start(add=True)

            @pl.when(ci + 6 < CHUNKS)
            def _next_idx():
                _issue_idx(ci + 6, islot)
        return _
    lax.fori_loop(0, CHUNKS // 6, ring, 0)

    for u in range(3):
        ci_tail = CHUNKS - 3 + u
        _wait_scatters(ci_tail % 6, ci_tail % DEPTH)

    plsc.subcore_barrier()

    pltpu.sync_copy(s_sh.at[pl.ds(sid * 640, 640)],
                    s_out.at[cid, pl.ds(sid * 640, 640)])

    @pl.when(sid < 15)
    def _pub_main():
        pltpu.sync_copy(acc_sh.at[pl.ds(sid * 640, 640)],
                        acc_out.at[cid, pl.ds(sid * 640, 640)])

    @pl.when(sid == 15)
    def _pub_tail():
        pltpu.sync_copy(acc_sh.at[pl.ds(9600, 400)],
                        acc_out.at[cid, pl.ds(9600, 400)])


def _edge_pass(h, alpha_s, alpha_d, sdp):
    mesh = plsc.VectorSubcoreMesh(core_axis_name="c", subcore_axis_name="s",
                                  num_cores=NC, num_subcores=NS)
    f = pl.kernel(
        _edge_body,
        out_type=(
            jax.ShapeDtypeStruct((NC, N, C), jnp.float32),
            jax.ShapeDtypeStruct((NC, 10240), jnp.float32),
        ),
        mesh=mesh,
        scratch_types=[
            pltpu.VMEM_SHARED((N, C), jnp.float32),
            pltpu.VMEM_SHARED((10240,), jnp.float32),
            pltpu.VMEM((K, C), jnp.float32),
            pltpu.VMEM((K, C), jnp.float32),
            pltpu.VMEM((K, C), jnp.float32),
            pltpu.VMEM((2, K), jnp.int32),
            pltpu.VMEM((2, K), jnp.int32),
            pltpu.VMEM((2, K), jnp.int32),
            pltpu.VMEM((2, K), jnp.int32),
            pltpu.VMEM((2, K), jnp.int32),
            pltpu.VMEM((2, K), jnp.int32),
            pltpu.VMEM((K,), jnp.float32),
            pltpu.VMEM((K,), jnp.float32),
            pltpu.VMEM((K,), jnp.float32),
            pltpu.VMEM((K,), jnp.float32),
            pltpu.VMEM((K,), jnp.float32),
            pltpu.VMEM((K,), jnp.float32),
            pltpu.VMEM((K,), jnp.float32),
            pltpu.VMEM((K,), jnp.float32),
            pltpu.VMEM((K,), jnp.float32),
            pltpu.SemaphoreType.DMA,
            pltpu.SemaphoreType.DMA,
            pltpu.SemaphoreType.DMA,
            pltpu.SemaphoreType.DMA,
            pltpu.SemaphoreType.DMA,
            pltpu.SemaphoreType.DMA,
            pltpu.SemaphoreType.DMA,
            pltpu.SemaphoreType.DMA,
            pltpu.SemaphoreType.DMA,
            pltpu.SemaphoreType.DMA,
            pltpu.SemaphoreType.DMA,
            pltpu.SemaphoreType.DMA,
            pltpu.SemaphoreType.DMA,
            pltpu.SemaphoreType.DMA,
            pltpu.SemaphoreType.DMA,
        ],
        compiler_params=pltpu.CompilerParams(needs_layout_passes=False),
    )
    return f(h, alpha_s, alpha_d, sdp,
             jnp.zeros((640, C), jnp.float32),
             jnp.zeros((10240,), jnp.float32))


def _k1_body(x_ref, w_ref, asw_ref, adw_ref, h_ref, als_ref, ald_ref):
    h = jnp.dot(x_ref[...], w_ref[...], preferred_element_type=jnp.float32)
    h_ref[...] = h
    als_ref[...] = h @ asw_ref[...]
    ald_ref[...] = h @ adw_ref[...]


def _k1(x, W, a_s, a_d):
    return pl.pallas_call(
        _k1_body,
        out_shape=(
            jax.ShapeDtypeStruct((N, C), jnp.float32),
            jax.ShapeDtypeStruct((N,), jnp.float32),
            jax.ShapeDtypeStruct((N,), jnp.float32),
        ),
    )(x, W, a_s, a_d)


def _combine(accp, sp, als, ald, h, b, gamma, beta):
    p_self = jnp.exp(_leaky(als + ald))
    self_w = (jnp.arange(N) >= NPE).astype(jnp.float32)
    p_self = p_self * self_w
    s_tot = sp[0, :N] + sp[1, :N] + p_self
    acc = accp[0] + accp[1] + p_self[:, None] * h
    g = acc / (s_tot + 1e-16)[:, None] + b
    mu = jnp.mean(g, axis=0)
    var = jnp.mean((g - mu) ** 2, axis=0)
    return (g - mu) / jnp.sqrt(var + 1e-5) * gamma + beta


def _k3_body(accp_ref, sp_ref, als_ref, ald_ref, h_ref, b_ref, g_ref, be_ref,
             w2_ref, asw_ref, adw_ref, h2_ref, als2_ref, ald2_ref):
    g = _combine(accp_ref[...], sp_ref[...], als_ref[...], ald_ref[...],
                 h_ref[...], b_ref[...], g_ref[...], be_ref[...])
    g = jnp.where(g > 0, g, jnp.exp(g) - 1.0)
    h2 = jnp.dot(g, w2_ref[...], preferred_element_type=jnp.float32)
    h2_ref[...] = h2
    als2_ref[...] = h2 @ asw_ref[...]
    ald2_ref[...] = h2 @ adw_ref[...]


def _k3(accp, sp, als, ald, h, b, gamma, beta, W2, a_s2, a_d2):
    return pl.pallas_call(
        _k3_body,
        out_shape=(
            jax.ShapeDtypeStruct((N, C), jnp.float32),
            jax.ShapeDtypeStruct((N,), jnp.float32),
            jax.ShapeDtypeStruct((N,), jnp.float32),
        ),
    )(accp, sp, als, ald, h, b, gamma, beta, W2, a_s2, a_d2)


def _k5_body(accp_ref, sp_ref, als_ref, ald_ref, h_ref, b_ref, g_ref, be_ref,
             x_ref, out_ref):
    g = _combine(accp_ref[...], sp_ref[...], als_ref[...], ald_ref[...],
                 h_ref[...], b_ref[...], g_ref[...], be_ref[...])
    g = jnp.maximum(g, 0.0)
    out_ref[...] = jnp.maximum(g + x_ref[...], 0.0)


def _k5(accp, sp, als, ald, h, b, gamma, beta, x):
    return pl.pallas_call(
        _k5_body,
        out_shape=jax.ShapeDtypeStruct((N, C), jnp.float32),
    )(accp, sp, als, ald, h, b, gamma, beta, x)


def kernel(x, edge_index, W1, att_src1, att_dst1, b1, gamma1, beta1,
           W2, att_src2, att_dst2, b2, gamma2, beta2):
    src = edge_index[0].astype(jnp.int32)
    dst = edge_index[1].astype(jnp.int32)
    pad_rng = jnp.arange(NPE, dtype=jnp.int32)
    srcp = jnp.concatenate([src, pad_rng]).reshape(NW * CHUNKS, 1, K)
    dstp = jnp.concatenate([dst, pad_rng]).reshape(NW * CHUNKS, 1, K)
    sdp = jnp.concatenate([srcp, dstp], axis=1)

    h1, als1, ald1 = _k1(x, W1, att_src1, att_dst1)
    accp1, sp1 = _edge_pass(h1, als1, ald1, sdp)
    h2, als2, ald2 = _k3(accp1, sp1, als1, ald1, h1, b1, gamma1, beta1,
                         W2, att_src2, att_dst2)
    accp2, sp2 = _edge_pass(h2, als2, ald2, sdp)
    return _k5(accp2, sp2, als2, ald2, h2, b2, gamma2, beta2, x)
